# baseline (speedup 1.0000x reference)
import functools

import jax
import jax.numpy as jnp
from jax import lax
from jax.experimental import pallas as pl
from jax.experimental.pallas import tpu as pltpu
from jax.experimental.pallas import tpu_sc as plsc

N = 10000
E = 320000
C = 128
K = 27

NC = 2
NS = 16
LANES = 16

TILES = NC * NS
CHUNK = 128
EDGES_PER_TILE = 10240
NCHUNK = EDGES_PER_TILE // CHUNK
STAGES = 2
CPS = NCHUNK // STAGES
E_PAD = TILES * EDGES_PER_TILE
ACC_ROWS = 10112
ZERO_ROWS = 32
ROWS_PER_TILE = ACC_ROWS // NS

E_ROWS = E // CHUNK
PAD_ROWS = (E_PAD - E) // CHUNK


def _matmul_body(x_ref, w_ref, src_ref, dst_ref, ki_ref, y_ref, g_ref, d_ref):
    y_ref[...] = jnp.dot(x_ref[...], w_ref[0], preferred_element_type=jnp.float32)

    @pl.when(pl.program_id(0) == 0)
    def _():
        g_ref[pl.ds(0, E_ROWS)] = ki_ref[...] * N + src_ref[...]
        d_ref[pl.ds(0, E_ROWS)] = dst_ref[...]
        flat = (
            jax.lax.broadcasted_iota(jnp.int32, (PAD_ROWS, CHUNK), 0) * CHUNK
            + jax.lax.broadcasted_iota(jnp.int32, (PAD_ROWS, CHUNK), 1)
        )
        g_ref[pl.ds(E_ROWS, PAD_ROWS)] = flat % (K * N)
        d_ref[pl.ds(E_ROWS, PAD_ROWS)] = N + flat % (ACC_ROWS - N)


def _compute_y(x, w, src2, dst2, ki2):
    return pl.pallas_call(
        _matmul_body,
        grid=(K,),
        in_specs=[
            pl.BlockSpec((N, C), lambda k: (0, 0)),
            pl.BlockSpec((1, C, C), lambda k: (k, 0, 0)),

            pl.BlockSpec((E_ROWS, CHUNK), lambda k: (0, 0)),
            pl.BlockSpec((E_ROWS, CHUNK), lambda k: (0, 0)),
            pl.BlockSpec((E_ROWS, CHUNK), lambda k: (0, 0)),
        ],
        out_specs=[
            pl.BlockSpec((N, C), lambda k: (k, 0)),
            pl.BlockSpec((E_ROWS + PAD_ROWS, CHUNK), lambda k: (0, 0)),
            pl.BlockSpec((E_ROWS + PAD_ROWS, CHUNK), lambda k: (0, 0)),
        ],
        out_shape=[
            jax.ShapeDtypeStruct((K * N, C), jnp.float32),
            jax.ShapeDtypeStruct((E_ROWS + PAD_ROWS, CHUNK), jnp.int32),
            jax.ShapeDtypeStruct((E_ROWS + PAD_ROWS, CHUNK), jnp.int32),
        ],
    )(x, w, src2, dst2, ki2)


def _sc_edge_accum(y, g3, d3):
    mesh = plsc.VectorSubcoreMesh(core_axis_name="c", subcore_axis_name="s")

    @functools.partial(
        pl.kernel,
        mesh=mesh,
        out_type=jax.ShapeDtypeStruct((NC, ACC_ROWS, C), jnp.float32),
        scratch_types=[
            pltpu.VMEM((CPS, CHUNK), jnp.int32),
            pltpu.VMEM((CPS, CHUNK), jnp.int32),
            pltpu.VMEM((CHUNK, C), jnp.float32),
            pltpu.VMEM((CHUNK, C), jnp.float32),
            pltpu.VMEM((ZERO_ROWS, C), jnp.float32),
            pltpu.VMEM_SHARED((ACC_ROWS, C), jnp.float32),
            pltpu.SemaphoreType.DMA,
            pltpu.SemaphoreType.DMA,
        ],
    )
    def sc_kernel(y_hbm, g_hbm, d_hbm, out_hbm, g_v, d_v, rows0, rows1, zbuf,
                  acc, sem0, sem1):
        cid = lax.axis_index("c")
        sid = lax.axis_index("s")
        wid = cid * NS + sid
        rows = (rows0, rows1)
        sems = (sem0, sem1)

        @pl.loop(0, ZERO_ROWS)
        def _(i):
            @pl.loop(0, C, step=LANES)
            def _(cc):
                zbuf[i, pl.ds(cc, LANES)] = jnp.zeros((LANES,), jnp.float32)

        pltpu.sync_copy(g_hbm.at[wid].at[0], g_v)
        pltpu.sync_copy(d_hbm.at[wid].at[0], d_v)
        pltpu.async_copy(y_hbm.at[g_v.at[0]], rows0, sem0)
        pltpu.async_copy(y_hbm.at[g_v.at[1]], rows1, sem1)

        @pl.loop(0, ROWS_PER_TILE - ZERO_ROWS, step=ZERO_ROWS)
        def _(r):
            pltpu.sync_copy(
                zbuf, acc.at[pl.ds(sid * ROWS_PER_TILE + r, ZERO_ROWS)]
            )
        pltpu.sync_copy(
            zbuf.at[pl.ds(0, 24)],
            acc.at[pl.ds(sid * ROWS_PER_TILE + ROWS_PER_TILE - 24, 24)],
        )

        plsc.subcore_barrier()

        for s in range(STAGES):
            if s > 0:
                pltpu.sync_copy(g_hbm.at[wid].at[s], g_v)
                pltpu.sync_copy(d_hbm.at[wid].at[s], d_v)
                pltpu.async_copy(y_hbm.at[g_v.at[0]], rows0, sem0)
                pltpu.async_copy(y_hbm.at[g_v.at[1]], rows1, sem1)

            @pl.loop(0, CPS, step=2)
            def _(j):
                for b in range(2):
                    pltpu.make_async_copy(
                        y_hbm.at[pl.ds(0, CHUNK)], rows[b], sems[b]
                    ).wait()
                    pltpu.sync_copy(rows[b], acc.at[d_v.at[j + b]], add=True)

                    @pl.when(j + b + 2 < CPS)
                    def _():
                        pltpu.async_copy(
                            y_hbm.at[g_v.at[j + b + 2]], rows[b], sems[b]
                        )

        plsc.subcore_barrier()

        pltpu.sync_copy(
            acc.at[pl.ds(sid * ROWS_PER_TILE, ROWS_PER_TILE)],
            out_hbm.at[cid].at[pl.ds(sid * ROWS_PER_TILE, ROWS_PER_TILE)],
        )

    return sc_kernel(y, g3, d3)


def _add_body(p_ref, o_ref):
    o_ref[...] = p_ref[0] + p_ref[1]


def _combine(p):
    return pl.pallas_call(
        _add_body,
        grid=(5,),
        in_specs=[pl.BlockSpec((NC, N // 5, C), lambda i: (0, i, 0))],
        out_specs=pl.BlockSpec((N // 5, C), lambda i: (i, 0)),
        out_shape=jax.ShapeDtypeStruct((N, C), jnp.float32),
    )(p)


@jax.jit
def kernel(x, edge_index, kernel_idx, kernel):
    src2 = edge_index[0].reshape(E_ROWS, CHUNK)
    dst2 = edge_index[1].reshape(E_ROWS, CHUNK)
    ki2 = kernel_idx.reshape(E_ROWS, CHUNK)

    x16 = x.astype(jnp.bfloat16)
    w16 = kernel.astype(jnp.bfloat16)
    y, g2, d2 = _compute_y(x16, w16, src2, dst2, ki2)
    g3 = g2.reshape(TILES, STAGES, CPS, CHUNK)
    d3 = d2.reshape(TILES, STAGES, CPS, CHUNK)
    p = _sc_edge_accum(y, g3, d3)
    return _combine(p)


# speedup vs baseline: 1.0612x; 1.0612x over previous
import functools

import jax
import jax.numpy as jnp
from jax import lax
from jax.experimental import pallas as pl
from jax.experimental.pallas import tpu as pltpu
from jax.experimental.pallas import tpu_sc as plsc

N = 10000
E = 320000
C = 128
K = 27

NC = 2
NS = 16
LANES = 16

TILES = NC * NS
CHUNK = 128
EDGES_PER_TILE = 10240
NCHUNK = EDGES_PER_TILE // CHUNK
STAGES = 2
CPS = NCHUNK // STAGES
E_PAD = TILES * EDGES_PER_TILE
ACC_ROWS = 10112
ZERO_ROWS = 32
ROWS_PER_TILE = ACC_ROWS // NS

E_ROWS = E // CHUNK
PAD_ROWS = (E_PAD - E) // CHUNK


def _matmul_body(x_ref, w_ref, src_ref, dst_ref, ki_ref, y_ref, g_ref, d_ref):
    y_ref[...] = jnp.dot(x_ref[...], w_ref[0], preferred_element_type=jnp.float32)

    @pl.when(pl.program_id(0) == 0)
    def _():
        g_ref[pl.ds(0, E_ROWS)] = ki_ref[...] * N + src_ref[...]
        d_ref[pl.ds(0, E_ROWS)] = dst_ref[...]
        flat = (
            jax.lax.broadcasted_iota(jnp.int32, (PAD_ROWS, CHUNK), 0) * CHUNK
            + jax.lax.broadcasted_iota(jnp.int32, (PAD_ROWS, CHUNK), 1)
        )
        g_ref[pl.ds(E_ROWS, PAD_ROWS)] = flat % (K * N)
        d_ref[pl.ds(E_ROWS, PAD_ROWS)] = N + flat % (ACC_ROWS - N)


def _compute_y(x, w, src2, dst2, ki2):
    return pl.pallas_call(
        _matmul_body,
        grid=(K,),
        in_specs=[
            pl.BlockSpec((N, C), lambda k: (0, 0)),
            pl.BlockSpec((1, C, C), lambda k: (k, 0, 0)),
            pl.BlockSpec((E_ROWS, CHUNK), lambda k: (0, 0)),
            pl.BlockSpec((E_ROWS, CHUNK), lambda k: (0, 0)),
            pl.BlockSpec((E_ROWS, CHUNK), lambda k: (0, 0)),
        ],
        out_specs=[
            pl.BlockSpec((N, C), lambda k: (k, 0)),
            pl.BlockSpec((E_ROWS + PAD_ROWS, CHUNK), lambda k: (0, 0)),
            pl.BlockSpec((E_ROWS + PAD_ROWS, CHUNK), lambda k: (0, 0)),
        ],
        out_shape=[
            jax.ShapeDtypeStruct((K * N, C), jnp.float32),
            jax.ShapeDtypeStruct((E_ROWS + PAD_ROWS, CHUNK), jnp.int32),
            jax.ShapeDtypeStruct((E_ROWS + PAD_ROWS, CHUNK), jnp.int32),
        ],
        compiler_params=pltpu.CompilerParams(
            dimension_semantics=("parallel",),
        ),
    )(x, w, src2, dst2, ki2)


def _sc_edge_accum(y, g3, d3):
    mesh = plsc.VectorSubcoreMesh(core_axis_name="c", subcore_axis_name="s")

    @functools.partial(
        pl.kernel,
        mesh=mesh,
        out_type=jax.ShapeDtypeStruct((NC, ACC_ROWS, C), jnp.float32),
        scratch_types=[
            pltpu.VMEM((CPS, CHUNK), jnp.int32),
            pltpu.VMEM((CPS, CHUNK), jnp.int32),
            pltpu.VMEM((CHUNK, C), jnp.float32),
            pltpu.VMEM((CHUNK, C), jnp.float32),
            pltpu.VMEM((ZERO_ROWS, C), jnp.float32),
            pltpu.VMEM_SHARED((ACC_ROWS, C), jnp.float32),
            pltpu.SemaphoreType.DMA,
            pltpu.SemaphoreType.DMA,
        ],
    )
    def sc_kernel(y_hbm, g_hbm, d_hbm, out_hbm, g_v, d_v, rows0, rows1, zbuf,
                  acc, sem0, sem1):
        cid = lax.axis_index("c")
        sid = lax.axis_index("s")
        wid = cid * NS + sid
        rows = (rows0, rows1)
        sems = (sem0, sem1)

        @pl.loop(0, ZERO_ROWS)
        def _(i):
            @pl.loop(0, C, step=LANES)
            def _(cc):
                zbuf[i, pl.ds(cc, LANES)] = jnp.zeros((LANES,), jnp.float32)

        pltpu.sync_copy(g_hbm.at[wid].at[0], g_v)
        pltpu.sync_copy(d_hbm.at[wid].at[0], d_v)
        pltpu.async_copy(y_hbm.at[g_v.at[0]], rows0, sem0)
        pltpu.async_copy(y_hbm.at[g_v.at[1]], rows1, sem1)

        @pl.loop(0, ROWS_PER_TILE - ZERO_ROWS, step=ZERO_ROWS)
        def _(r):
            pltpu.sync_copy(
                zbuf, acc.at[pl.ds(sid * ROWS_PER_TILE + r, ZERO_ROWS)]
            )
        pltpu.sync_copy(
            zbuf.at[pl.ds(0, 24)],
            acc.at[pl.ds(sid * ROWS_PER_TILE + ROWS_PER_TILE - 24, 24)],
        )

        plsc.subcore_barrier()

        for s in range(STAGES):
            if s > 0:
                pltpu.sync_copy(g_hbm.at[wid].at[s], g_v)
                pltpu.sync_copy(d_hbm.at[wid].at[s], d_v)
                pltpu.async_copy(y_hbm.at[g_v.at[0]], rows0, sem0)
                pltpu.async_copy(y_hbm.at[g_v.at[1]], rows1, sem1)

            @pl.loop(0, CPS, step=2)
            def _(j):
                for b in range(2):
                    pltpu.make_async_copy(
                        y_hbm.at[pl.ds(0, CHUNK)], rows[b], sems[b]
                    ).wait()
                    pltpu.sync_copy(rows[b], acc.at[d_v.at[j + b]], add=True)

                    @pl.when(j + b + 2 < CPS)
                    def _():
                        pltpu.async_copy(
                            y_hbm.at[g_v.at[j + b + 2]], rows[b], sems[b]
                        )

        plsc.subcore_barrier()

        pltpu.sync_copy(
            acc.at[pl.ds(sid * ROWS_PER_TILE, ROWS_PER_TILE)],
            out_hbm.at[cid].at[pl.ds(sid * ROWS_PER_TILE, ROWS_PER_TILE)],
        )

    return sc_kernel(y, g3, d3)


def _add_body(p_ref, o_ref):
    o_ref[...] = p_ref[0] + p_ref[1]


def _combine(p):
    return pl.pallas_call(
        _add_body,
        grid=(5,),
        in_specs=[pl.BlockSpec((NC, N // 5, C), lambda i: (0, i, 0))],
        out_specs=pl.BlockSpec((N // 5, C), lambda i: (i, 0)),
        out_shape=jax.ShapeDtypeStruct((N, C), jnp.float32),
    )(p)


@jax.jit
def kernel(x, edge_index, kernel_idx, kernel):
    src2 = edge_index[0].reshape(E_ROWS, CHUNK)
    dst2 = edge_index[1].reshape(E_ROWS, CHUNK)
    ki2 = kernel_idx.reshape(E_ROWS, CHUNK)

    y, g2, d2 = _compute_y(x, kernel, src2, dst2, ki2)
    g3 = g2.reshape(TILES, STAGES, CPS, CHUNK)
    d3 = d2.reshape(TILES, STAGES, CPS, CHUNK)
    p = _sc_edge_accum(y, g3, d3)
    return _combine(p)
